# baseline (speedup 1.0000x reference)
import jax
import jax.numpy as jnp
from jax.experimental import pallas as pl

_LANES = 128
_ROWS = 5000


def _body(nu_blocks, u_ref, it_ref, ego_ref, ua_ref, ia_ref):
    i = pl.program_id(0)
    u = u_ref[...]
    it = it_ref[...]
    ua_ref[...] = u
    ia_ref[...] = it

    @pl.when(i < nu_blocks)
    def _():
        ego_ref[...] = u

    @pl.when(i >= nu_blocks)
    def _():
        ego_ref[...] = it


def kernel(user_weight, item_weight):
    n_users, emb = user_weight.shape
    n_items, _ = item_weight.shape

    u_rows = n_users * emb // _LANES
    i_rows = n_items * emb // _LANES
    uf = user_weight.reshape(u_rows, _LANES)
    itf = item_weight.reshape(i_rows, _LANES)

    nu = u_rows // _ROWS
    ni = i_rows // _ROWS

    blk = (_ROWS, _LANES)
    ego_f, ua_f, ia_f = pl.pallas_call(
        lambda *refs: _body(nu, *refs),
        grid=(nu + ni,),
        in_specs=[
            pl.BlockSpec(blk, lambda i: (jnp.minimum(i, nu - 1), 0)),
            pl.BlockSpec(blk, lambda i: (jnp.maximum(i - nu, 0), 0)),
        ],
        out_specs=[
            pl.BlockSpec(blk, lambda i: (i, 0)),
            pl.BlockSpec(blk, lambda i: (jnp.minimum(i, nu - 1), 0)),
            pl.BlockSpec(blk, lambda i: (jnp.maximum(i - nu, 0), 0)),
        ],
        out_shape=[
            jax.ShapeDtypeStruct((u_rows + i_rows, _LANES), jnp.float32),
            jax.ShapeDtypeStruct((u_rows, _LANES), jnp.float32),
            jax.ShapeDtypeStruct((i_rows, _LANES), jnp.float32),
        ],
    )(uf, itf)

    return (
        ua_f.reshape(n_users, emb),
        ia_f.reshape(n_items, emb),
        ego_f.reshape(n_users + n_items, emb),
    )


# speedup vs baseline: 1.2894x; 1.2894x over previous
import jax
import jax.numpy as jnp
from jax.experimental import pallas as pl

_ROWS = 10000


def _body(nu_blocks, u_ref, it_ref, ego_ref, ua_ref, ia_ref):
    i = pl.program_id(0)
    u = u_ref[...]
    it = it_ref[...]
    ua_ref[...] = u
    ia_ref[...] = it

    @pl.when(i < nu_blocks)
    def _():
        ego_ref[...] = u

    @pl.when(i >= nu_blocks)
    def _():
        ego_ref[...] = it


def kernel(user_weight, item_weight):
    n_users, emb = user_weight.shape
    n_items, _ = item_weight.shape

    nu = n_users // _ROWS
    ni = n_items // _ROWS

    blk = (_ROWS, emb)
    ego, ua, ia = pl.pallas_call(
        lambda *refs: _body(nu, *refs),
        grid=(nu + ni,),
        in_specs=[
            pl.BlockSpec(blk, lambda i: (jnp.minimum(i, nu - 1), 0)),
            pl.BlockSpec(blk, lambda i: (jnp.maximum(i - nu, 0), 0)),
        ],
        out_specs=[
            pl.BlockSpec(blk, lambda i: (i, 0)),
            pl.BlockSpec(blk, lambda i: (jnp.minimum(i, nu - 1), 0)),
            pl.BlockSpec(blk, lambda i: (jnp.maximum(i - nu, 0), 0)),
        ],
        out_shape=[
            jax.ShapeDtypeStruct((n_users + n_items, emb), jnp.float32),
            jax.ShapeDtypeStruct((n_users, emb), jnp.float32),
            jax.ShapeDtypeStruct((n_items, emb), jnp.float32),
        ],
    )(user_weight, item_weight)

    return (ua, ia, ego)
